# Initial kernel scaffold; baseline (speedup 1.0000x reference)
import jax
import jax.numpy as jnp
from jax.experimental import pallas as pl


def kernel(user_emb, item_emb, edge_user, edge_item):
    raise NotImplementedError("write your pallas kernel here")


# speedup vs baseline: 11.9563x; 11.9563x over previous
import functools

import jax
import jax.numpy as jnp
from jax import lax
from jax.experimental import pallas as pl
from jax.experimental.pallas import tpu as pltpu
from jax.experimental.pallas import tpu_sc as plsc

U = 50000
D = 32
E = 800000
L_LAYERS = 3

NS = 16
CHUNK = 128
EPT_RAW = E // NS
NCHUNK = -(-EPT_RAW // CHUNK)
EPT = NCHUNK * CHUNK
E_PAD = EPT * NS
PAD_IDX = U
TROWS = 50048
OPT = TROWS // NS
ZC = 25
AROWS = NS * ZC * CHUNK
DW = 16

_MESH = plsc.VectorSubcoreMesh(core_axis_name="c", subcore_axis_name="s")


def _zero_fill(buf, width):
    def body(i, _):
        for w0 in range(0, width, 16):
            buf[i, pl.ds(w0, 16)] = jnp.zeros((16,), jnp.float32)
        return 0
    lax.fori_loop(0, CHUNK, body, 0)


def _one_fill(buf):
    def body(i, _):
        buf[i] = jnp.ones((16,), jnp.float32)
        return 0
    lax.fori_loop(0, CHUNK, body, 0)


@functools.partial(
    pl.kernel,
    out_type=(
        jax.ShapeDtypeStruct((TROWS, D), jnp.float32),
        jax.ShapeDtypeStruct((TROWS, D), jnp.float32),
    ),
    mesh=_MESH,
    scratch_types=[
        pltpu.VMEM((CHUNK,), jnp.int32),
        pltpu.VMEM((CHUNK,), jnp.int32),
        pltpu.VMEM((CHUNK, D), jnp.float32),
        pltpu.VMEM((CHUNK, D), jnp.float32),
        pltpu.VMEM_SHARED((AROWS, D), jnp.float32),
        pltpu.SemaphoreType.DMA,
    ],
    compiler_params=pltpu.CompilerParams(use_tc_tiling_on_sc=False),
)
def _propagate(hu, hi, eu, ei, outu, outi, cidx, ridx, gbuf, zbuf, accum, sem):
    c = lax.axis_index("c")
    s = lax.axis_index("s")

    _zero_fill(zbuf, D)

    def zc_body(i, _):
        pltpu.sync_copy(zbuf, accum.at[pl.ds((s * ZC + i) * CHUNK, CHUNK)])
        return 0
    lax.fori_loop(0, ZC, zc_body, 0)
    plsc.subcore_barrier()

    def run(tab, gih, sih, outh):
        base = s * EPT

        def body(j, _):
            off = base + j * CHUNK
            pltpu.sync_copy(gih.at[pl.ds(off, CHUNK)], cidx)
            pltpu.sync_copy(sih.at[pl.ds(off, CHUNK)], ridx)
            pltpu.async_copy(tab.at[cidx], gbuf, sem).wait()
            pltpu.sync_copy(gbuf, accum.at[ridx], add=True)
            return 0
        lax.fori_loop(0, NCHUNK, body, 0)
        plsc.subcore_barrier()
        pltpu.sync_copy(accum.at[pl.ds(s * OPT, OPT)], outh.at[pl.ds(s * OPT, OPT)])

    pl.when(c == 0)(lambda: run(hi, ei, eu, outu))
    pl.when(c == 1)(lambda: run(hu, eu, ei, outi))


@functools.partial(
    pl.kernel,
    out_type=(
        jax.ShapeDtypeStruct((TROWS, DW), jnp.float32),
        jax.ShapeDtypeStruct((TROWS, DW), jnp.float32),
    ),
    mesh=_MESH,
    scratch_types=[
        pltpu.VMEM((CHUNK,), jnp.int32),
        pltpu.VMEM((CHUNK, DW), jnp.float32),
        pltpu.VMEM((CHUNK, DW), jnp.float32),
        pltpu.VMEM_SHARED((AROWS, DW), jnp.float32),
        pltpu.SemaphoreType.DMA,
    ],
    compiler_params=pltpu.CompilerParams(use_tc_tiling_on_sc=False),
)
def _degree(eu, ei, outu, outi, sidx, ones, zbuf, dacc, sem):
    c = lax.axis_index("c")
    s = lax.axis_index("s")

    _one_fill(ones)
    _zero_fill(zbuf, DW)

    def zc_body(i, _):
        pltpu.sync_copy(zbuf, dacc.at[pl.ds((s * ZC + i) * CHUNK, CHUNK)])
        return 0
    lax.fori_loop(0, ZC, zc_body, 0)
    plsc.subcore_barrier()

    def run(sih, outh):
        base = s * EPT

        def body(j, _):
            pltpu.sync_copy(sih.at[pl.ds(base + j * CHUNK, CHUNK)], sidx)
            pltpu.sync_copy(ones, dacc.at[sidx], add=True)
            return 0
        lax.fori_loop(0, NCHUNK, body, 0)
        plsc.subcore_barrier()
        pltpu.sync_copy(dacc.at[pl.ds(s * OPT, OPT)], outh.at[pl.ds(s * OPT, OPT)])

    pl.when(c == 0)(lambda: run(eu, outu))
    pl.when(c == 1)(lambda: run(ei, outi))


def _pad_edges(e):
    r = e.reshape(NS, EPT_RAW)
    p = jnp.full((NS, EPT - EPT_RAW), PAD_IDX, e.dtype)
    return jnp.concatenate([r, p], axis=1).reshape(-1)


def kernel(user_emb, item_emb, edge_user, edge_item):
    eu = _pad_edges(edge_user.astype(jnp.int32))
    ei = _pad_edges(edge_item.astype(jnp.int32))

    pad = ((0, TROWS - U), (0, 0))
    hu = jnp.pad(user_emb, pad)
    hi = jnp.pad(item_emb, pad)

    degu, degi = _degree(eu, ei)
    dinv_u = jnp.where(degu[:, :1] > 0, 1.0 / degu[:, :1], 0.0)
    dinv_i = jnp.where(degi[:, :1] > 0, 1.0 / degi[:, :1], 0.0)

    acc_u, acc_i = hu, hi
    for _ in range(L_LAYERS):
        su, si = _propagate(hu, hi, eu, ei)
        hu = dinv_u * su
        hi = dinv_i * si
        acc_u = acc_u + hu
        acc_i = acc_i + hi

    scale = 1.0 / (L_LAYERS + 1)
    return jnp.concatenate([acc_u[:U] * scale, acc_i[:U] * scale], axis=0)


# speedup vs baseline: 19.6372x; 1.6424x over previous
import functools

import jax
import jax.numpy as jnp
from jax import lax
from jax.experimental import pallas as pl
from jax.experimental.pallas import tpu as pltpu
from jax.experimental.pallas import tpu_sc as plsc

U = 50000
D = 32
E = 800000
L_LAYERS = 3

NS = 16
CHUNK = 128
NCHUNK = 392
NSTORE = NCHUNK + 2
EPT = NSTORE * CHUNK
EPT_RAW = E // NS
GPAD = 0
SPAD = U
AROWS = 50048
ZPT = AROWS // NS
OPT = U // NS
DW = 8

_MESH = plsc.VectorSubcoreMesh(core_axis_name="c", subcore_axis_name="s")


@functools.partial(
    pl.kernel,
    out_type=(
        jax.ShapeDtypeStruct((U, D), jnp.float32),
        jax.ShapeDtypeStruct((U, D), jnp.float32),
    ),
    mesh=_MESH,
    scratch_types=[
        pltpu.VMEM((CHUNK,), jnp.int32),
        pltpu.VMEM((CHUNK,), jnp.int32),
        pltpu.VMEM((CHUNK,), jnp.int32),
        pltpu.VMEM((CHUNK,), jnp.int32),
        pltpu.VMEM((CHUNK, D), jnp.float32),
        pltpu.VMEM((CHUNK, D), jnp.float32),
        pltpu.VMEM_SHARED((AROWS, D), jnp.float32),
        pltpu.SemaphoreType.DMA,
        pltpu.SemaphoreType.DMA,
        pltpu.SemaphoreType.DMA,
        pltpu.SemaphoreType.DMA,
    ],
    compiler_params=pltpu.CompilerParams(use_tc_tiling_on_sc=False),
)
def _propagate(hu, hi, eug, eus, eig, eis, zrows, outu, outi,
               cidx0, cidx1, ridx0, ridx1, gbuf0, gbuf1, accum,
               isem0, isem1, gsem0, gsem1):
    c = lax.axis_index("c")
    s = lax.axis_index("s")

    def run(tab, gih, sih, outh):
        def load_idx(j, cbuf, rbuf, sem):
            pltpu.async_copy(gih.at[s, j], cbuf, sem)
            pltpu.async_copy(sih.at[s, j], rbuf, sem)

        def wait_idx(j, cbuf, rbuf, sem):
            pltpu.make_async_copy(gih.at[s, j], cbuf, sem).wait()
            pltpu.make_async_copy(sih.at[s, j], rbuf, sem).wait()

        load_idx(0, cidx0, ridx0, isem0)
        pltpu.sync_copy(zrows, accum.at[pl.ds(s * ZPT, ZPT)])
        wait_idx(0, cidx0, ridx0, isem0)
        plsc.subcore_barrier()

        pltpu.async_copy(tab.at[cidx0], gbuf0, gsem0)
        load_idx(1, cidx1, ridx1, isem1)

        def outer(j0, _):
            j = j0 * 2
            wait_idx(j + 1, cidx1, ridx1, isem1)
            pltpu.make_async_copy(tab.at[cidx0], gbuf0, gsem0).wait()
            pltpu.sync_copy(gbuf0, accum.at[ridx0], add=True)
            load_idx(j + 2, cidx0, ridx0, isem0)
            pltpu.async_copy(tab.at[cidx1], gbuf1, gsem1)
            wait_idx(j + 2, cidx0, ridx0, isem0)
            pltpu.make_async_copy(tab.at[cidx1], gbuf1, gsem1).wait()
            pltpu.sync_copy(gbuf1, accum.at[ridx1], add=True)
            load_idx(j + 3, cidx1, ridx1, isem1)
            pltpu.async_copy(tab.at[cidx0], gbuf0, gsem0)
            return 0
        lax.fori_loop(0, NCHUNK // 2, outer, 0)
        wait_idx(NCHUNK + 1, cidx1, ridx1, isem1)
        pltpu.make_async_copy(tab.at[cidx0], gbuf0, gsem0).wait()

        plsc.subcore_barrier()
        pltpu.sync_copy(accum.at[pl.ds(s * OPT, OPT)], outh.at[pl.ds(s * OPT, OPT)])

    pl.when(c == 0)(lambda: run(hi, eig, eus, outu))
    pl.when(c == 1)(lambda: run(hu, eug, eis, outi))


@functools.partial(
    pl.kernel,
    out_type=(
        jax.ShapeDtypeStruct((U, DW), jnp.float32),
        jax.ShapeDtypeStruct((U, DW), jnp.float32),
    ),
    mesh=_MESH,
    scratch_types=[
        pltpu.VMEM((NSTORE, CHUNK), jnp.int32),
        pltpu.VMEM((CHUNK, DW), jnp.float32),
        pltpu.VMEM_SHARED((AROWS, DW), jnp.float32),
        pltpu.SemaphoreType.DMA,
        pltpu.SemaphoreType.DMA,
    ],
    compiler_params=pltpu.CompilerParams(use_tc_tiling_on_sc=False),
)
def _degree(eus, eis, ones_in, zrows, outu, outi, ridx, ones, dacc, isem, ssem):
    c = lax.axis_index("c")
    s = lax.axis_index("s")

    def run(sih, outh):
        icp = pltpu.async_copy(sih.at[s], ridx, isem)
        ocp = pltpu.async_copy(ones_in, ones, isem)
        pltpu.sync_copy(zrows, dacc.at[pl.ds(s * ZPT, ZPT)])
        icp.wait()
        ocp.wait()
        plsc.subcore_barrier()

        def outer(j0, _):
            j = j0 * 4
            for b in range(4):
                pltpu.async_copy(ones, dacc.at[ridx.at[j + b]], ssem, add=True)
            for b in range(4):
                pltpu.make_async_copy(ones, dacc.at[ridx.at[j + b]], ssem).wait()
            return 0
        lax.fori_loop(0, NCHUNK // 4, outer, 0)

        plsc.subcore_barrier()
        pltpu.sync_copy(dacc.at[pl.ds(s * OPT, OPT)], outh.at[pl.ds(s * OPT, OPT)])

    pl.when(c == 0)(lambda: run(eus, outu))
    pl.when(c == 1)(lambda: run(eis, outi))


def _pad_edges(e, pad_val):
    r = e.reshape(NS, EPT_RAW)
    p = jnp.full((NS, EPT - EPT_RAW), pad_val, e.dtype)
    return jnp.concatenate([r, p], axis=1).reshape(NS, NSTORE, CHUNK)


def kernel(user_emb, item_emb, edge_user, edge_item):
    eu = edge_user.astype(jnp.int32)
    ei = edge_item.astype(jnp.int32)
    eug, eus = _pad_edges(eu, GPAD), _pad_edges(eu, SPAD)
    eig, eis = _pad_edges(ei, GPAD), _pad_edges(ei, SPAD)

    zrows = jnp.zeros((ZPT, D), jnp.float32)
    zrows_d = jnp.zeros((ZPT, DW), jnp.float32)
    ones_in = jnp.ones((CHUNK, DW), jnp.float32)

    degu, degi = _degree(eus, eis, ones_in, zrows_d)
    dinv_u = jnp.where(degu[:, :1] > 0, 1.0 / degu[:, :1], 0.0)
    dinv_i = jnp.where(degi[:, :1] > 0, 1.0 / degi[:, :1], 0.0)

    hu, hi = user_emb, item_emb
    acc_u, acc_i = hu, hi
    for _ in range(L_LAYERS):
        su, si = _propagate(hu, hi, eug, eus, eig, eis, zrows)
        hu = dinv_u * su
        hi = dinv_i * si
        acc_u = acc_u + hu
        acc_i = acc_i + hi

    scale = 1.0 / (L_LAYERS + 1)
    return jnp.concatenate([acc_u * scale, acc_i * scale], axis=0)


# speedup vs baseline: 26.7099x; 1.3602x over previous
import functools

import jax
import jax.numpy as jnp
from jax import lax
from jax.experimental import pallas as pl
from jax.experimental.pallas import tpu as pltpu
from jax.experimental.pallas import tpu_sc as plsc

U = 50000
D = 32
E = 800000
L_LAYERS = 3

NS = 16
CHUNK = 128
NCHUNK = 392
NSTORE = NCHUNK + 4
EPT = NSTORE * CHUNK
EPT_RAW = E // NS
GPAD = 0
SPAD = U
AROWS = 50048
ZPT = AROWS // NS
OPT = U // NS
DW = 8

_MESH = plsc.VectorSubcoreMesh(core_axis_name="c", subcore_axis_name="s")


@functools.partial(
    pl.kernel,
    out_type=(
        jax.ShapeDtypeStruct((U, D), jnp.float32),
        jax.ShapeDtypeStruct((U, D), jnp.float32),
    ),
    mesh=_MESH,
    scratch_types=[
        [pltpu.VMEM((CHUNK,), jnp.int32) for _ in range(4)],
        [pltpu.VMEM((CHUNK,), jnp.int32) for _ in range(4)],
        [pltpu.VMEM((CHUNK, D), jnp.float32) for _ in range(2)],
        pltpu.VMEM_SHARED((AROWS, D), jnp.float32),
        [pltpu.SemaphoreType.DMA for _ in range(4)],
        [pltpu.SemaphoreType.DMA for _ in range(2)],
    ],
    compiler_params=pltpu.CompilerParams(use_tc_tiling_on_sc=False),
)
def _propagate(hu, hi, eug, eus, eig, eis, zrows, outu, outi,
               cidx, ridx, gbuf, accum, isem, gsem):
    c = lax.axis_index("c")
    s = lax.axis_index("s")

    def run(tab, gih, sih, outh):
        def fire_idx(j, b):
            pltpu.async_copy(gih.at[s, j], cidx[b], isem[b])
            pltpu.async_copy(sih.at[s, j], ridx[b], isem[b])

        def wait_idx(j, b):
            pltpu.make_async_copy(gih.at[s, j], cidx[b], isem[b]).wait()
            pltpu.make_async_copy(sih.at[s, j], ridx[b], isem[b]).wait()

        for b in range(4):
            fire_idx(b, b)
        pltpu.sync_copy(zrows, accum.at[pl.ds(s * ZPT, ZPT)])
        plsc.subcore_barrier()
        wait_idx(0, 0)
        pltpu.async_copy(tab.at[cidx[0]], gbuf[0], gsem[0])
        wait_idx(1, 1)
        pltpu.async_copy(tab.at[cidx[1]], gbuf[1], gsem[1])

        def outer(j0, _):
            j = j0 * 4
            for b in range(4):
                g = b % 2
                pltpu.make_async_copy(tab.at[cidx[b]], gbuf[g], gsem[g]).wait()
                pltpu.sync_copy(gbuf[g], accum.at[ridx[b]], add=True)
                fire_idx(j + b + 4, b)
                wait_idx(j + b + 2, (b + 2) % 4)
                pltpu.async_copy(tab.at[cidx[(b + 2) % 4]], gbuf[g], gsem[g])
            return 0
        lax.fori_loop(0, NCHUNK // 4, outer, 0)
        pltpu.make_async_copy(tab.at[cidx[0]], gbuf[0], gsem[0]).wait()
        pltpu.make_async_copy(tab.at[cidx[1]], gbuf[1], gsem[1]).wait()
        wait_idx(NCHUNK + 2, 2)
        wait_idx(NCHUNK + 3, 3)

        plsc.subcore_barrier()
        pltpu.sync_copy(accum.at[pl.ds(s * OPT, OPT)], outh.at[pl.ds(s * OPT, OPT)])

    pl.when(c == 0)(lambda: run(hi, eig, eus, outu))
    pl.when(c == 1)(lambda: run(hu, eug, eis, outi))


@functools.partial(
    pl.kernel,
    out_type=(
        jax.ShapeDtypeStruct((U, DW), jnp.float32),
        jax.ShapeDtypeStruct((U, DW), jnp.float32),
    ),
    mesh=_MESH,
    scratch_types=[
        pltpu.VMEM((NSTORE, CHUNK), jnp.int32),
        pltpu.VMEM((CHUNK, DW), jnp.float32),
        pltpu.VMEM_SHARED((AROWS, DW), jnp.float32),
        pltpu.SemaphoreType.DMA,
        pltpu.SemaphoreType.DMA,
    ],
    compiler_params=pltpu.CompilerParams(use_tc_tiling_on_sc=False),
)
def _degree(eus, eis, ones_in, zrows, outu, outi, ridx, ones, dacc, isem, ssem):
    c = lax.axis_index("c")
    s = lax.axis_index("s")

    def run(sih, outh):
        icp = pltpu.async_copy(sih.at[s], ridx, isem)
        ocp = pltpu.async_copy(ones_in, ones, isem)
        pltpu.sync_copy(zrows, dacc.at[pl.ds(s * ZPT, ZPT)])
        icp.wait()
        ocp.wait()
        plsc.subcore_barrier()

        def outer(j0, _):
            j = j0 * 4
            for b in range(4):
                pltpu.async_copy(ones, dacc.at[ridx.at[j + b]], ssem, add=True)
            for b in range(4):
                pltpu.make_async_copy(ones, dacc.at[ridx.at[j + b]], ssem).wait()
            return 0
        lax.fori_loop(0, NCHUNK // 4, outer, 0)

        plsc.subcore_barrier()
        pltpu.sync_copy(dacc.at[pl.ds(s * OPT, OPT)], outh.at[pl.ds(s * OPT, OPT)])

    pl.when(c == 0)(lambda: run(eus, outu))
    pl.when(c == 1)(lambda: run(eis, outi))


def _pad_edges(e, pad_val):
    r = e.reshape(NS, EPT_RAW)
    p = jnp.full((NS, EPT - EPT_RAW), pad_val, e.dtype)
    return jnp.concatenate([r, p], axis=1).reshape(NS, NSTORE, CHUNK)


def kernel(user_emb, item_emb, edge_user, edge_item):
    eu = edge_user.astype(jnp.int32)
    ei = edge_item.astype(jnp.int32)
    eug, eus = _pad_edges(eu, GPAD), _pad_edges(eu, SPAD)
    eig, eis = _pad_edges(ei, GPAD), _pad_edges(ei, SPAD)

    zrows = jnp.zeros((ZPT, D), jnp.float32)
    zrows_d = jnp.zeros((ZPT, DW), jnp.float32)
    ones_in = jnp.ones((CHUNK, DW), jnp.float32)

    degu, degi = _degree(eus, eis, ones_in, zrows_d)
    dinv_u = jnp.where(degu[:, :1] > 0, 1.0 / degu[:, :1], 0.0)
    dinv_i = jnp.where(degi[:, :1] > 0, 1.0 / degi[:, :1], 0.0)

    hu, hi = user_emb, item_emb
    acc_u, acc_i = hu, hi
    for _ in range(L_LAYERS):
        su, si = _propagate(hu, hi, eug, eus, eig, eis, zrows)
        hu = dinv_u * su
        hi = dinv_i * si
        acc_u = acc_u + hu
        acc_i = acc_i + hi

    scale = 1.0 / (L_LAYERS + 1)
    return jnp.concatenate([acc_u * scale, acc_i * scale], axis=0)
